# baseline (speedup 1.0000x reference)
import jax
import jax.numpy as jnp
from jax.experimental import pallas as pl
from jax.experimental.pallas import tpu as pltpu

NSTREAMS = 2
BLOCK_M = 512


def _router_block(*refs):
    h_refs = refs[:NSTREAMS]
    w_ref = refs[NSTREAMS]
    out_hbm = refs[NSTREAMS + 1]
    o_vmem = refs[NSTREAMS + 2]
    sems = refs[NSTREAMS + 3]
    i = pl.program_id(0)
    n = pl.num_programs(0)
    half = out_hbm.shape[0] // NSTREAMS
    w = w_ref[...]

    def probs(h):
        logits = jax.lax.dot_general(
            h, w, (((1,), (1,)), ((), ())), preferred_element_type=jnp.float32
        )
        m = jnp.max(logits, axis=-1, keepdims=True)
        e = jnp.exp(logits - m)
        return e / jnp.sum(e, axis=-1, keepdims=True)

    def out_copy(step, s):
        return pltpu.make_async_copy(
            o_vmem.at[s],
            out_hbm.at[pl.ds(s * half + step * BLOCK_M, BLOCK_M), :],
            sems.at[s],
        )

    for s in range(NSTREAMS):
        @pl.when(i >= 1)
        def _(s=s):
            out_copy(i - 1, s).wait()

        o_vmem[s] = probs(h_refs[s][...])
        out_copy(i, s).start()

    @pl.when(i == n - 1)
    def _():
        for s in range(NSTREAMS):
            out_copy(i, s).wait()


def kernel(hidden_states, gate_weight):
    n_tokens, hidden = hidden_states.shape
    n_experts = gate_weight.shape[0]
    per_stream = n_tokens // BLOCK_M // NSTREAMS
    grid = (per_stream,)
    h_specs = [
        pl.BlockSpec((BLOCK_M, hidden), lambda i, s=s, p=per_stream: (i + s * p, 0))
        for s in range(NSTREAMS)
    ]
    return pl.pallas_call(
        _router_block,
        grid=grid,
        in_specs=h_specs + [pl.BlockSpec((n_experts, hidden), lambda i: (0, 0))],
        out_specs=pl.BlockSpec(memory_space=pltpu.MemorySpace.HBM),
        out_shape=jax.ShapeDtypeStruct((n_tokens, n_experts), jnp.float32),
        scratch_shapes=[
            pltpu.VMEM((NSTREAMS, BLOCK_M, n_experts), jnp.float32),
            pltpu.SemaphoreType.DMA((NSTREAMS,)),
        ],
        compiler_params=pltpu.CompilerParams(
            dimension_semantics=("arbitrary",),
        ),
    )(*([hidden_states] * NSTREAMS), gate_weight)


# speedup vs baseline: 1.1208x; 1.1208x over previous
import jax
import jax.numpy as jnp
from jax.experimental import pallas as pl
from jax.experimental.pallas import tpu as pltpu

NSTREAMS = 4
BLOCK_M = 256


def _router_block(*refs):
    h_refs = refs[:NSTREAMS]
    w_ref = refs[NSTREAMS]
    out_ref = refs[NSTREAMS + 1]
    w = w_ref[...]

    def probs(h):
        logits = jax.lax.dot_general(
            h, w, (((1,), (1,)), ((), ())), preferred_element_type=jnp.float32
        )
        m = jnp.max(logits, axis=-1, keepdims=True)
        e = jnp.exp(logits - m)
        return e / jnp.sum(e, axis=-1, keepdims=True)

    for s in range(NSTREAMS):
        out_ref[s] = probs(h_refs[s][...])


def kernel(hidden_states, gate_weight):
    n_tokens, hidden = hidden_states.shape
    n_experts = gate_weight.shape[0]
    per_stream = n_tokens // BLOCK_M // NSTREAMS
    grid = (per_stream,)
    h_specs = [
        pl.BlockSpec((BLOCK_M, hidden), lambda i, s=s, p=per_stream: (i + s * p, 0))
        for s in range(NSTREAMS)
    ]
    out = pl.pallas_call(
        _router_block,
        grid=grid,
        in_specs=h_specs + [pl.BlockSpec((n_experts, hidden), lambda i: (0, 0))],
        out_specs=pl.BlockSpec((NSTREAMS, BLOCK_M, n_experts), lambda i: (0, i, 0)),
        out_shape=jax.ShapeDtypeStruct(
            (NSTREAMS, n_tokens // NSTREAMS, n_experts), jnp.float32
        ),
        compiler_params=pltpu.CompilerParams(
            dimension_semantics=("arbitrary",),
        ),
    )(*([hidden_states] * NSTREAMS), gate_weight)
    return out.reshape(n_tokens, n_experts)
